# baseline (speedup 1.0000x reference)
import functools

import jax
import jax.numpy as jnp
from jax import lax
from jax.experimental import pallas as pl
from jax.experimental.pallas import tpu as pltpu
from jax.experimental.pallas import tpu_sc as plsc

B, S, H = 4, 2048, 768
VOCAB = 1000
EPS = 1e-12
L = 16
NC, NS = 2, 16
NW = NC * NS
PPW = S // NW
NCHUNK = H // L


def _rsqrt(x):
    i = lax.bitcast_convert_type(x, jnp.int32)
    i = jnp.int32(0x5F3759DF) - (i >> 1)
    y = lax.bitcast_convert_type(i, jnp.float32)
    for _ in range(4):
        y = y * (1.5 - 0.5 * x * y * y)
    return y


def _sc_body(ids_hbm, word_hbm, pos_hbm, tok_hbm, gamma_hbm, beta_hbm,
             out_hbm, base_v, word_v, idx_v, tok_v, gamma_v, beta_v, sem):
    wid = lax.axis_index("s") * NC + lax.axis_index("c")
    pos0 = wid * PPW

    pltpu.sync_copy(pos_hbm.at[pl.ds(pos0, PPW)], base_v)
    pltpu.sync_copy(tok_hbm.at[0], tok_v)
    pltpu.sync_copy(gamma_hbm, gamma_v)
    pltpu.sync_copy(beta_hbm, beta_v)

    def fold_tok(r, _):
        def fold_chunk(j, _):
            sl = pl.ds(j * L, L)
            base_v[r, sl] = base_v[r, sl] + tok_v[sl]
            return 0
        return lax.fori_loop(0, NCHUNK, fold_chunk, 0)
    lax.fori_loop(0, PPW, fold_tok, 0)

    for b in range(B):
        flat0 = b * S + pos0
        pltpu.sync_copy(ids_hbm.at[pl.ds(flat0, PPW)], idx_v)
        pltpu.async_copy(word_hbm.at[idx_v], word_v, sem).wait()

        def do_row(r, _):
            def stats(j, carry):
                acc, acc2 = carry
                sl = pl.ds(j * L, L)
                y = word_v[r, sl] + base_v[r, sl]
                word_v[r, sl] = y
                return acc + y, acc2 + y * y

            acc, acc2 = lax.fori_loop(
                0, NCHUNK, stats,
                (jnp.zeros((L,), jnp.float32), jnp.zeros((L,), jnp.float32)))
            s1 = acc[0]
            s2 = acc2[0]
            for i in range(1, L):
                s1 = s1 + acc[i]
                s2 = s2 + acc2[i]
            mean = s1 * (1.0 / H)
            var = s2 * (1.0 / H) - mean * mean
            inv = _rsqrt(var + EPS)
            shift = -mean * inv

            def norm(j, _):
                sl = pl.ds(j * L, L)
                y = word_v[r, sl]
                word_v[r, sl] = (y * inv + shift) * gamma_v[sl] + beta_v[sl]
                return 0
            return lax.fori_loop(0, NCHUNK, norm, 0)

        lax.fori_loop(0, PPW, do_row, 0)
        pltpu.sync_copy(word_v, out_hbm.at[pl.ds(flat0, PPW)])


@jax.jit
def _embed_ln(ids_flat, word_table, pos_table, tok_table, ln_gamma, ln_beta):
    mesh = plsc.VectorSubcoreMesh(core_axis_name="c", subcore_axis_name="s")
    out = pl.kernel(
        _sc_body,
        out_type=jax.ShapeDtypeStruct((B * S, H), jnp.float32),
        mesh=mesh,
        scratch_types=[
            pltpu.VMEM((PPW, H), jnp.float32),
            pltpu.VMEM((PPW, H), jnp.float32),
            pltpu.VMEM((PPW,), jnp.int32),
            pltpu.VMEM((H,), jnp.float32),
            pltpu.VMEM((H,), jnp.float32),
            pltpu.VMEM((H,), jnp.float32),
            pltpu.SemaphoreType.DMA,
        ],
    )(ids_flat, word_table, pos_table, tok_table, ln_gamma, ln_beta)
    return out


def kernel(input_ids, word_table, pos_table, tok_table, ln_gamma, ln_beta):
    ids_flat = input_ids.reshape(-1)
    out = _embed_ln(ids_flat, word_table, pos_table, tok_table,
                    ln_gamma, ln_beta)
    return out.reshape(B, S, H)


# speedup vs baseline: 3.4139x; 3.4139x over previous
import functools

import jax
import jax.numpy as jnp
from jax import lax
from jax.experimental import pallas as pl
from jax.experimental.pallas import tpu as pltpu
from jax.experimental.pallas import tpu_sc as plsc

B, S, H = 4, 2048, 768
EPS = 1e-12
NC, NS = 2, 16
NW = NC * NS
RPW = B * S // NW
GC = 64
NCH = RPW // GC
BR = 512


def _sc_gather_body(ids_hbm, word_hbm, out_hbm, idx_v, buf0, buf1,
                    sem_g0, sem_g1, sem_w0, sem_w1):
    wid = lax.axis_index("s") * NC + lax.axis_index("c")
    base = wid * RPW
    pltpu.sync_copy(ids_hbm.at[pl.ds(base, RPW)], idx_v)

    bufs = (buf0, buf1)
    gsems = (sem_g0, sem_g1)
    wsems = (sem_w0, sem_w1)

    def gather(c):
        return pltpu.async_copy(
            word_hbm.at[idx_v.at[pl.ds(c * GC, GC)]], bufs[c % 2],
            gsems[c % 2])
    def write(c):
        return pltpu.async_copy(
            bufs[c % 2], out_hbm.at[pl.ds(base + c * GC, GC)], wsems[c % 2])

    g = [None] * NCH
    w = [None] * NCH
    g[0] = gather(0)
    for c in range(NCH):
        g[c].wait()
        if c + 1 < NCH:
            if c >= 1:
                w[c - 1].wait()
            g[c + 1] = gather(c + 1)
        w[c] = write(c)
    w[NCH - 2].wait()
    w[NCH - 1].wait()


def _tc_ln_body(g_ref, pos_ref, tok_ref, gamma_ref, beta_ref, o_ref):
    y = g_ref[...] + pos_ref[...] + tok_ref[...]
    mean = jnp.mean(y, axis=-1, keepdims=True)
    c = y - mean
    var = jnp.mean(c * c, axis=-1, keepdims=True)
    o_ref[...] = c * lax.rsqrt(var + EPS) * gamma_ref[...] + beta_ref[...]


@jax.jit
def _embed_ln(ids_flat, word_table, pos_table, tok_table, ln_gamma, ln_beta):
    mesh = plsc.VectorSubcoreMesh(core_axis_name="c", subcore_axis_name="s")
    gathered = pl.kernel(
        _sc_gather_body,
        out_type=jax.ShapeDtypeStruct((B * S, H), jnp.float32),
        mesh=mesh,
        scratch_types=[
            pltpu.VMEM((RPW,), jnp.int32),
            pltpu.VMEM((GC, H), jnp.float32),
            pltpu.VMEM((GC, H), jnp.float32),
            pltpu.SemaphoreType.DMA,
            pltpu.SemaphoreType.DMA,
            pltpu.SemaphoreType.DMA,
            pltpu.SemaphoreType.DMA,
        ],
    )(ids_flat, word_table)

    nblk = B * S // BR
    sblk = S // BR
    out = pl.pallas_call(
        _tc_ln_body,
        grid=(nblk,),
        in_specs=[
            pl.BlockSpec((BR, H), lambda i: (i, 0)),
            pl.BlockSpec((BR, H), lambda i: (i % sblk, 0)),
            pl.BlockSpec((1, H), lambda i: (0, 0)),
            pl.BlockSpec((1, H), lambda i: (0, 0)),
            pl.BlockSpec((1, H), lambda i: (0, 0)),
        ],
        out_specs=pl.BlockSpec((BR, H), lambda i: (i, 0)),
        out_shape=jax.ShapeDtypeStruct((B * S, H), jnp.float32),
    )(gathered, pos_table[:S], tok_table[:1], ln_gamma[None, :],
      ln_beta[None, :])
    return out


def kernel(input_ids, word_table, pos_table, tok_table, ln_gamma, ln_beta):
    ids_flat = input_ids.reshape(-1)
    out = _embed_ln(ids_flat, word_table, pos_table, tok_table,
                    ln_gamma, ln_beta)
    return out.reshape(B, S, H)


# speedup vs baseline: 3.7427x; 1.0963x over previous
import functools

import jax
import jax.numpy as jnp
from jax import lax
from jax.experimental import pallas as pl
from jax.experimental.pallas import tpu as pltpu
from jax.experimental.pallas import tpu_sc as plsc

B, S, H = 4, 2048, 768
EPS = 1e-12
NC, NS = 2, 16
NW = NC * NS
RPW = B * S // NW
GC = 32
NCH = RPW // GC
NBUF = 4
BR = 512
SBLK = S // BR


def _sc_gather_body(ids_hbm, word_hbm, out_hbm, idx_v, *bufs_and_sems):
    bufs = bufs_and_sems[:NBUF]
    gsems = bufs_and_sems[NBUF:2 * NBUF]
    wsems = bufs_and_sems[2 * NBUF:3 * NBUF]
    wid = lax.axis_index("s") * NC + lax.axis_index("c")
    base = wid * RPW
    pltpu.sync_copy(ids_hbm.at[pl.ds(base, RPW)], idx_v)

    def gather(c):
        return pltpu.async_copy(
            word_hbm.at[idx_v.at[pl.ds(c * GC, GC)]], bufs[c % NBUF],
            gsems[c % NBUF])

    def write(c):
        return pltpu.async_copy(
            bufs[c % NBUF], out_hbm.at[pl.ds(base + c * GC, GC)],
            wsems[c % NBUF])

    g = [None] * NCH
    w = [None] * NCH
    for c in range(NBUF):
        g[c] = gather(c)
    for c in range(NCH):
        g[c].wait()
        w[c] = write(c)
        nxt = c + NBUF
        if nxt < NCH:
            w[c].wait()
            g[nxt] = gather(nxt)
    for c in range(NCH - NBUF, NCH):
        w[c].wait()


def _tc_ln_body(g_ref, pos_ref, tok_ref, gamma_ref, beta_ref, o_ref):
    y = g_ref[...] + pos_ref[...] + tok_ref[0:1, :]
    mean = jnp.mean(y, axis=-1, keepdims=True)
    c = y - mean
    var = jnp.mean(c * c, axis=-1, keepdims=True)
    o_ref[...] = c * lax.rsqrt(var + EPS) * gamma_ref[...] + beta_ref[...]


@jax.jit
def _embed_ln(ids_flat, word_table, pos_table, tok_table, ln_gamma, ln_beta):
    mesh = plsc.VectorSubcoreMesh(core_axis_name="c", subcore_axis_name="s")
    gathered = pl.kernel(
        _sc_gather_body,
        out_type=jax.ShapeDtypeStruct((B * S, H), jnp.float32),
        mesh=mesh,
        scratch_types=(
            [pltpu.VMEM((RPW,), jnp.int32)]
            + [pltpu.VMEM((GC, H), jnp.float32)] * NBUF
            + [pltpu.SemaphoreType.DMA] * (2 * NBUF)
        ),
    )(ids_flat, word_table)

    out = pl.pallas_call(
        _tc_ln_body,
        grid=(SBLK, B),
        in_specs=[
            pl.BlockSpec((BR, H), lambda p, b: (b * SBLK + p, 0)),
            pl.BlockSpec((BR, H), lambda p, b: (p, 0)),
            pl.BlockSpec((2, H), lambda p, b: (0, 0)),
            pl.BlockSpec((1, H), lambda p, b: (0, 0)),
            pl.BlockSpec((1, H), lambda p, b: (0, 0)),
        ],
        out_specs=pl.BlockSpec((BR, H), lambda p, b: (b * SBLK + p, 0)),
        out_shape=jax.ShapeDtypeStruct((B * S, H), jnp.float32),
    )(gathered, pos_table, tok_table, ln_gamma[None, :], ln_beta[None, :])
    return out


def kernel(input_ids, word_table, pos_table, tok_table, ln_gamma, ln_beta):
    ids_flat = input_ids.reshape(-1)
    out = _embed_ln(ids_flat, word_table, pos_table, tok_table,
                    ln_gamma, ln_beta)
    return out.reshape(B, S, H)


# speedup vs baseline: 3.9438x; 1.0537x over previous
import functools

import jax
import jax.numpy as jnp
from jax import lax
from jax.experimental import pallas as pl
from jax.experimental.pallas import tpu as pltpu
from jax.experimental.pallas import tpu_sc as plsc

B, S, H = 4, 2048, 768
EPS = 1e-12
NC, NS = 2, 16
NW = NC * NS
RPW = B * S // NW
GC = 32
NCH = RPW // GC
NBUF = 5
BR = 1024
SBLK = S // BR


def _sc_gather_body(ids_hbm, word_hbm, out_hbm, idx_v, *bufs_and_sems):
    bufs = bufs_and_sems[:NBUF]
    gsems = bufs_and_sems[NBUF:2 * NBUF]
    wsems = bufs_and_sems[2 * NBUF:3 * NBUF]
    wid = lax.axis_index("s") * NC + lax.axis_index("c")
    base = wid * RPW
    pltpu.sync_copy(ids_hbm.at[pl.ds(base, RPW)], idx_v)

    def gather(c):
        return pltpu.async_copy(
            word_hbm.at[idx_v.at[pl.ds(c * GC, GC)]], bufs[c % NBUF],
            gsems[c % NBUF])

    def write(c):
        return pltpu.async_copy(
            bufs[c % NBUF], out_hbm.at[pl.ds(base + c * GC, GC)],
            wsems[c % NBUF])

    g = [None] * NCH
    w = [None] * NCH
    for c in range(NBUF):
        g[c] = gather(c)
    for c in range(NCH):
        g[c].wait()
        w[c] = write(c)
        nxt = c + NBUF
        if nxt < NCH:
            w[c].wait()
            g[nxt] = gather(nxt)
    for c in range(NCH - NBUF, NCH):
        w[c].wait()


def _tc_ln_body(g_ref, pos_ref, tok_ref, gamma_ref, beta_ref, o_ref):
    y = g_ref[...] + pos_ref[...] + tok_ref[0:1, :]
    mean = jnp.mean(y, axis=-1, keepdims=True)
    c = y - mean
    var = jnp.mean(c * c, axis=-1, keepdims=True)
    o_ref[...] = c * lax.rsqrt(var + EPS) * gamma_ref[...] + beta_ref[...]


@jax.jit
def _embed_ln(ids_flat, word_table, pos_table, tok_table, ln_gamma, ln_beta):
    mesh = plsc.VectorSubcoreMesh(core_axis_name="c", subcore_axis_name="s")
    gathered = pl.kernel(
        _sc_gather_body,
        out_type=jax.ShapeDtypeStruct((B * S, H), jnp.float32),
        mesh=mesh,
        scratch_types=(
            [pltpu.VMEM((RPW,), jnp.int32)]
            + [pltpu.VMEM((GC, H), jnp.float32)] * NBUF
            + [pltpu.SemaphoreType.DMA] * (2 * NBUF)
        ),
    )(ids_flat, word_table)

    out = pl.pallas_call(
        _tc_ln_body,
        grid=(SBLK, B),
        in_specs=[
            pl.BlockSpec((BR, H), lambda p, b: (b * SBLK + p, 0)),
            pl.BlockSpec((BR, H), lambda p, b: (p, 0)),
            pl.BlockSpec((2, H), lambda p, b: (0, 0)),
            pl.BlockSpec((1, H), lambda p, b: (0, 0)),
            pl.BlockSpec((1, H), lambda p, b: (0, 0)),
        ],
        out_specs=pl.BlockSpec((BR, H), lambda p, b: (b * SBLK + p, 0)),
        out_shape=jax.ShapeDtypeStruct((B * S, H), jnp.float32),
    )(gathered, pos_table, tok_table, ln_gamma[None, :], ln_beta[None, :])
    return out


def kernel(input_ids, word_table, pos_table, tok_table, ln_gamma, ln_beta):
    ids_flat = input_ids.reshape(-1)
    out = _embed_ln(ids_flat, word_table, pos_table, tok_table,
                    ln_gamma, ln_beta)
    return out.reshape(B, S, H)


# speedup vs baseline: 4.0030x; 1.0150x over previous
import functools

import jax
import jax.numpy as jnp
from jax import lax
from jax.experimental import pallas as pl
from jax.experimental.pallas import tpu as pltpu
from jax.experimental.pallas import tpu_sc as plsc

B, S, H = 4, 2048, 768
EPS = 1e-12
NC, NS = 2, 16
NW = NC * NS
RPW = B * S // NW
GC = 32
NCH = RPW // GC
NBUF = 5
BR = 2048
SBLK = S // BR


def _sc_gather_body(ids_hbm, word_hbm, out_hbm, idx_v, *bufs_and_sems):
    bufs = bufs_and_sems[:NBUF]
    gsems = bufs_and_sems[NBUF:2 * NBUF]
    wsems = bufs_and_sems[2 * NBUF:3 * NBUF]
    wid = lax.axis_index("s") * NC + lax.axis_index("c")
    base = wid * RPW
    pltpu.sync_copy(ids_hbm.at[pl.ds(base, RPW)], idx_v)

    def gather(c):
        return pltpu.async_copy(
            word_hbm.at[idx_v.at[pl.ds(c * GC, GC)]], bufs[c % NBUF],
            gsems[c % NBUF])

    def write(c):
        return pltpu.async_copy(
            bufs[c % NBUF], out_hbm.at[pl.ds(base + c * GC, GC)],
            wsems[c % NBUF])

    g = [None] * NCH
    w = [None] * NCH
    for c in range(NBUF):
        g[c] = gather(c)
    for c in range(NCH):
        g[c].wait()
        w[c] = write(c)
        nxt = c + NBUF
        if nxt < NCH:
            w[c].wait()
            g[nxt] = gather(nxt)
    for c in range(NCH - NBUF, NCH):
        w[c].wait()


def _tc_ln_body(g_ref, pos_ref, tok_ref, gamma_ref, beta_ref, o_ref):
    y = g_ref[...] + pos_ref[...] + tok_ref[0:1, :]
    mean = jnp.mean(y, axis=-1, keepdims=True)
    c = y - mean
    var = jnp.mean(c * c, axis=-1, keepdims=True)
    o_ref[...] = c * lax.rsqrt(var + EPS) * gamma_ref[...] + beta_ref[...]


@jax.jit
def _embed_ln(ids_flat, word_table, pos_table, tok_table, ln_gamma, ln_beta):
    mesh = plsc.VectorSubcoreMesh(core_axis_name="c", subcore_axis_name="s")
    gathered = pl.kernel(
        _sc_gather_body,
        out_type=jax.ShapeDtypeStruct((B * S, H), jnp.float32),
        mesh=mesh,
        scratch_types=(
            [pltpu.VMEM((RPW,), jnp.int32)]
            + [pltpu.VMEM((GC, H), jnp.float32)] * NBUF
            + [pltpu.SemaphoreType.DMA] * (2 * NBUF)
        ),
    )(ids_flat, word_table)

    out = pl.pallas_call(
        _tc_ln_body,
        grid=(SBLK, B),
        in_specs=[
            pl.BlockSpec((BR, H), lambda p, b: (b * SBLK + p, 0)),
            pl.BlockSpec((BR, H), lambda p, b: (p, 0)),
            pl.BlockSpec((2, H), lambda p, b: (0, 0)),
            pl.BlockSpec((1, H), lambda p, b: (0, 0)),
            pl.BlockSpec((1, H), lambda p, b: (0, 0)),
        ],
        out_specs=pl.BlockSpec((BR, H), lambda p, b: (b * SBLK + p, 0)),
        out_shape=jax.ShapeDtypeStruct((B * S, H), jnp.float32),
    )(gathered, pos_table, tok_table, ln_gamma[None, :], ln_beta[None, :])
    return out


def kernel(input_ids, word_table, pos_table, tok_table, ln_gamma, ln_beta):
    ids_flat = input_ids.reshape(-1)
    out = _embed_ln(ids_flat, word_table, pos_table, tok_table,
                    ln_gamma, ln_beta)
    return out.reshape(B, S, H)


# speedup vs baseline: 4.6988x; 1.1738x over previous
import functools

import jax
import jax.numpy as jnp
from jax import lax
from jax.experimental import pallas as pl
from jax.experimental.pallas import tpu as pltpu
from jax.experimental.pallas import tpu_sc as plsc

B, S, H = 4, 2048, 768
HW = H // 2
EPS = 1e-12
NC, NS = 2, 16
NW = NC * NS
RPW = B * S // NW
GC = 32
NCH = RPW // GC
BR = 2048


def _sc_gather_body(ids_hbm, word_hbm, out_hbm, idx_v, *bufs_and_sems):
    bufs = bufs_and_sems[:NCH]
    gsems = bufs_and_sems[NCH:2 * NCH]
    wsems = bufs_and_sems[2 * NCH:3 * NCH]
    wid = lax.axis_index("s") * NC + lax.axis_index("c")
    base = wid * RPW
    pltpu.sync_copy(ids_hbm.at[pl.ds(base, RPW)], idx_v)

    g = [
        pltpu.async_copy(
            word_hbm.at[idx_v.at[pl.ds(c * GC, GC)]], bufs[c], gsems[c])
        for c in range(NCH)
    ]
    w = []
    for c in range(NCH):
        g[c].wait()
        w.append(pltpu.async_copy(
            bufs[c], out_hbm.at[pl.ds(base + c * GC, GC)], wsems[c]))
    for h in w:
        h.wait()


def _tc_ln_body(g_ref, pos_ref, tok_ref, gamma_ref, beta_ref, o_ref):
    w = g_ref[...]
    lo = lax.bitcast_convert_type(w << 16, jnp.float32)
    hi = lax.bitcast_convert_type(w & jnp.int32(-65536), jnp.float32)
    y = jnp.concatenate([lo, hi], axis=-1)
    y = y + pos_ref[...] + tok_ref[0:1, :]
    mean = jnp.mean(y, axis=-1, keepdims=True)
    c = y - mean
    var = jnp.mean(c * c, axis=-1, keepdims=True)
    o_ref[...] = c * lax.rsqrt(var + EPS) * gamma_ref[...] + beta_ref[...]


@jax.jit
def _embed_ln(ids_flat, word_table, pos_table, tok_table, ln_gamma, ln_beta):
    word_b16 = word_table.astype(jnp.bfloat16)
    word_i32 = lax.bitcast_convert_type(
        jnp.stack([word_b16[:, :HW], word_b16[:, HW:]], axis=-1), jnp.int32)

    mesh = plsc.VectorSubcoreMesh(core_axis_name="c", subcore_axis_name="s")
    gathered = pl.kernel(
        _sc_gather_body,
        out_type=jax.ShapeDtypeStruct((B * S, HW), jnp.int32),
        mesh=mesh,
        scratch_types=(
            [pltpu.VMEM((RPW,), jnp.int32)]
            + [pltpu.VMEM((GC, HW), jnp.int32)] * NCH
            + [pltpu.SemaphoreType.DMA] * (2 * NCH)
        ),
    )(ids_flat, word_i32)

    nblk = B * S // BR
    sblk = max(S // BR, 1)
    out = pl.pallas_call(
        _tc_ln_body,
        grid=(nblk,),
        in_specs=[
            pl.BlockSpec((BR, HW), lambda i: (i, 0)),
            pl.BlockSpec((BR, H), lambda i: (i % sblk, 0)),
            pl.BlockSpec((2, H), lambda i: (0, 0)),
            pl.BlockSpec((1, H), lambda i: (0, 0)),
            pl.BlockSpec((1, H), lambda i: (0, 0)),
        ],
        out_specs=pl.BlockSpec((BR, H), lambda i: (i, 0)),
        out_shape=jax.ShapeDtypeStruct((B * S, H), jnp.float32),
    )(gathered, pos_table, tok_table, ln_gamma[None, :], ln_beta[None, :])
    return out


def kernel(input_ids, word_table, pos_table, tok_table, ln_gamma, ln_beta):
    ids_flat = input_ids.reshape(-1)
    out = _embed_ln(ids_flat, word_table, pos_table, tok_table,
                    ln_gamma, ln_beta)
    return out.reshape(B, S, H)


# speedup vs baseline: 4.7286x; 1.0063x over previous
import functools

import jax
import jax.numpy as jnp
from jax import lax
from jax.experimental import pallas as pl
from jax.experimental.pallas import tpu as pltpu
from jax.experimental.pallas import tpu_sc as plsc

B, S, H = 4, 2048, 768
HW = H // 2
EPS = 1e-12
NC, NS = 2, 16
NW = NC * NS
RPW = B * S // NW
GC = 32
NCH = RPW // GC
BR = 2048


def _sc_gather_body(ids_hbm, word_hbm, out_hbm, idx_v, *bufs_and_sems):
    bufs = bufs_and_sems[:NCH]
    gsems = bufs_and_sems[NCH:2 * NCH]
    wsems = bufs_and_sems[2 * NCH:3 * NCH]
    wid = lax.axis_index("s") * NC + lax.axis_index("c")
    base = wid * RPW
    pltpu.sync_copy(ids_hbm.at[pl.ds(base, RPW)], idx_v)

    g = [
        pltpu.async_copy(
            word_hbm.at[idx_v.at[pl.ds(c * GC, GC)]], bufs[c], gsems[c])
        for c in range(NCH)
    ]
    w = []
    for c in range(NCH):
        g[c].wait()
        w.append(pltpu.async_copy(
            bufs[c], out_hbm.at[pl.ds(base + c * GC, GC)], wsems[c]))
    for h in w:
        h.wait()


def _tc_ln_body(g_ref, pos_ref, tok_ref, gamma_ref, beta_ref, o_ref):
    w = g_ref[...]
    lo = lax.bitcast_convert_type(w << 16, jnp.float32)
    hi = lax.bitcast_convert_type(w & jnp.int32(-65536), jnp.float32)
    y = jnp.concatenate([lo, hi], axis=-1)
    y = y + pos_ref[...] + tok_ref[0:1, :]
    mean = jnp.mean(y, axis=-1, keepdims=True)
    c = y - mean
    var = jnp.mean(c * c, axis=-1, keepdims=True)
    o_ref[...] = c * lax.rsqrt(var + EPS) * gamma_ref[...] + beta_ref[...]


@jax.jit
def _embed_ln(ids_flat, word_table, pos_table, tok_table, ln_gamma, ln_beta):
    wu = lax.bitcast_convert_type(word_table, jnp.uint32)
    lo, hi = wu[:, :HW], wu[:, HW:]
    rlo = (lo + jnp.uint32(0x7FFF) + ((lo >> 16) & jnp.uint32(1))) >> 16
    rhi = (hi + jnp.uint32(0x7FFF) + ((hi >> 16) & jnp.uint32(1))) \
        & jnp.uint32(0xFFFF0000)
    word_i32 = lax.bitcast_convert_type(rlo | rhi, jnp.int32)

    mesh = plsc.VectorSubcoreMesh(core_axis_name="c", subcore_axis_name="s")
    gathered = pl.kernel(
        _sc_gather_body,
        out_type=jax.ShapeDtypeStruct((B * S, HW), jnp.int32),
        mesh=mesh,
        scratch_types=(
            [pltpu.VMEM((RPW,), jnp.int32)]
            + [pltpu.VMEM((GC, HW), jnp.int32)] * NCH
            + [pltpu.SemaphoreType.DMA] * (2 * NCH)
        ),
    )(ids_flat, word_i32)

    nblk = B * S // BR
    sblk = max(S // BR, 1)
    out = pl.pallas_call(
        _tc_ln_body,
        grid=(nblk,),
        in_specs=[
            pl.BlockSpec((BR, HW), lambda i: (i, 0)),
            pl.BlockSpec((BR, H), lambda i: (i % sblk, 0)),
            pl.BlockSpec((2, H), lambda i: (0, 0)),
            pl.BlockSpec((1, H), lambda i: (0, 0)),
            pl.BlockSpec((1, H), lambda i: (0, 0)),
        ],
        out_specs=pl.BlockSpec((BR, H), lambda i: (i, 0)),
        out_shape=jax.ShapeDtypeStruct((B * S, H), jnp.float32),
    )(gathered, pos_table, tok_table, ln_gamma[None, :], ln_beta[None, :])
    return out


def kernel(input_ids, word_table, pos_table, tok_table, ln_gamma, ln_beta):
    ids_flat = input_ids.reshape(-1)
    out = _embed_ln(ids_flat, word_table, pos_table, tok_table,
                    ln_gamma, ln_beta)
    return out.reshape(B, S, H)
